# baseline (speedup 1.0000x reference)
import functools

import jax
import jax.numpy as jnp
from jax.experimental import pallas as pl
from jax.experimental.pallas import tpu as pltpu

TAU = 0.05


def _moe_block_kernel(h_ref, wr_ref, br_ref, w1_ref, b1_ref, w2_ref, b2_ref,
                      out_ref, *, n_experts, expert_dim):
    h = h_ref[...]
    h_bf = h.astype(jnp.bfloat16)
    logits = jax.lax.dot_general(
        h_bf, wr_ref[...].astype(jnp.bfloat16), (((1,), (0,)), ((), ())),
        preferred_element_type=jnp.float32) + br_ref[...]
    logits = logits - jnp.max(logits, axis=1, keepdims=True)
    expw = jnp.exp(logits)
    weights = expw / jnp.sum(expw, axis=1, keepdims=True)
    weights = jnp.where(weights > TAU, weights, 0.0)

    hidden = jax.lax.dot_general(
        h_bf, w1_ref[...], (((1,), (0,)), ((), ())),
        preferred_element_type=jnp.float32)
    hidden = (hidden + b1_ref[...]).astype(jnp.bfloat16)
    act = jax.nn.gelu(hidden)
    w_bf = weights.astype(jnp.bfloat16)
    scaled = jnp.concatenate(
        [act[:, e * expert_dim:(e + 1) * expert_dim] * w_bf[:, e:e + 1]
         for e in range(n_experts)], axis=1)
    y = jax.lax.dot_general(
        scaled, w2_ref[...], (((1,), (0,)), ((), ())),
        preferred_element_type=jnp.float32)
    y = y + jax.lax.dot_general(
        weights, b2_ref[...], (((1,), (0,)), ((), ())),
        precision=jax.lax.Precision.HIGHEST,
        preferred_element_type=jnp.float32)
    out_ref[...] = y


@jax.jit
def kernel(h, Wr, br, W1, b1, W2, b2):
    T, D = h.shape
    E = Wr.shape[1]
    F = W1.shape[2]
    BT = 256
    w1_flat = W1.transpose(1, 0, 2).reshape(D, E * F).astype(jnp.bfloat16)
    w2_flat = W2.reshape(E * F, D).astype(jnp.bfloat16)
    b1_flat = b1.reshape(1, E * F)
    br2 = br.reshape(1, E)
    grid = (T // BT,)
    return pl.pallas_call(
        functools.partial(_moe_block_kernel, n_experts=E, expert_dim=F),
        grid=grid,
        in_specs=[
            pl.BlockSpec((BT, D), lambda i: (i, 0)),
            pl.BlockSpec((D, E), lambda i: (0, 0)),
            pl.BlockSpec((1, E), lambda i: (0, 0)),
            pl.BlockSpec((D, E * F), lambda i: (0, 0)),
            pl.BlockSpec((1, E * F), lambda i: (0, 0)),
            pl.BlockSpec((E * F, D), lambda i: (0, 0)),
            pl.BlockSpec((E, D), lambda i: (0, 0)),
        ],
        out_specs=pl.BlockSpec((BT, D), lambda i: (i, 0)),
        out_shape=jax.ShapeDtypeStruct((T, D), jnp.float32),
        compiler_params=pltpu.CompilerParams(
            dimension_semantics=("arbitrary",),
        ),
    )(h, Wr, br2, w1_flat, b1_flat, w2_flat, b2)
